# baseline (speedup 1.0000x reference)
import functools

import jax
import jax.numpy as jnp
from jax import lax
from jax.experimental import pallas as pl
from jax.experimental.pallas import tpu as pltpu
from jax.experimental.pallas import tpu_sc as plsc

_SEQ_LENS = (2048, 1792, 1536, 1280, 1024, 768, 512, 256)
_B = 8
_MAX = 2048
_D = 1024
_NC = 2
_NS = 16
_NW = _NC * _NS
_ZROWS = 64


def _balanced_buckets():
    units = []
    for b, length in enumerate(_SEQ_LENS):
        units.append((2, b, 0, length))
        if length < _MAX:
            units.append((1, b, length, _MAX - length))
    total = sum(c * n for c, _, _, n in units)
    per = total / _NW
    buckets = [[] for _ in range(_NW)]
    w, acc = 0, 0.0
    for c, b, j0, n in units:
        while n > 0:
            if w == _NW - 1:
                take = n
            else:
                room = max(per - acc, 0.0)
                take = int(round(room / c / 8.0)) * 8
                take = max(8, min(take, n))
                if n - take < 8:
                    take = n
            buckets[w].append((c, b, j0, take))
            j0 += take
            n -= take
            acc += c * take
            if acc >= per - 1e-9 and w < _NW - 1:
                w += 1
                acc = 0.0
    return buckets


_BUCKETS = _balanced_buckets()


def _pad_body(x0, x1, x2, x3, x4, x5, x6, x7, out, zbuf, sem):
    xs = (x0, x1, x2, x3, x4, x5, x6, x7)
    wid = lax.axis_index("s") * _NC + lax.axis_index("c")

    def _zero_row(r, carry):
        for cc in range(_D // 16):
            zbuf[r, pl.ds(cc * 16, 16)] = jnp.zeros((16,), jnp.float32)
        return carry

    lax.fori_loop(0, _ZROWS, _zero_row, 0)

    for w, spans in enumerate(_BUCKETS):

        @pl.when(wid == w)
        def _(spans=spans):
            pending = []
            for c, b, j0, n in spans:
                if c == 2:
                    pending.append(
                        pltpu.async_copy(
                            xs[b].at[pl.ds(j0, n), :],
                            out.at[b, pl.ds(j0, n), :],
                            sem,
                        )
                    )
                else:
                    off = 0
                    while off < n:
                        m = min(_ZROWS, n - off)
                        pending.append(
                            pltpu.async_copy(
                                zbuf.at[pl.ds(0, m), :],
                                out.at[b, pl.ds(j0 + off, m), :],
                                sem,
                            )
                        )
                        off += m
            for p in pending:
                p.wait()


@functools.cache
def _get_padder():
    return pl.kernel(
        _pad_body,
        out_type=jax.ShapeDtypeStruct((_B, _MAX, _D), jnp.float32),
        mesh=plsc.VectorSubcoreMesh(
            core_axis_name="c",
            subcore_axis_name="s",
            num_cores=_NC,
            num_subcores=_NS,
        ),
        scratch_types=[
            pltpu.VMEM((_ZROWS, _D), jnp.float32),
            pltpu.SemaphoreType.DMA,
        ],
    )


def kernel(x0, x1, x2, x3, x4, x5, x6, x7):
    return _get_padder()(x0, x1, x2, x3, x4, x5, x6, x7)


# speedup vs baseline: 15.3846x; 15.3846x over previous
import functools

import jax
import jax.numpy as jnp
from jax import lax
from jax.experimental import pallas as pl
from jax.experimental.pallas import tpu as pltpu
from jax.experimental.pallas import tpu_sc as plsc

_SEQ_LENS = (2048, 1792, 1536, 1280, 1024, 768, 512, 256)
_B = 8
_MAX = 2048
_D = 1024
_NC = 2
_NS = 16
_NW = _NC * _NS
_ZROWS = 64
_CROWS = 24


def _balanced_buckets():
    units = []
    for b, length in enumerate(_SEQ_LENS):
        units.append((2, b, 0, length))
        if length < _MAX:
            units.append((1, b, length, _MAX - length))
    total = sum(c * n for c, _, _, n in units)
    per = total / _NW
    buckets = [[] for _ in range(_NW)]
    w, acc = 0, 0.0
    for c, b, j0, n in units:
        while n > 0:
            if w == _NW - 1:
                take = n
            else:
                room = max(per - acc, 0.0)
                take = int(round(room / c / 8.0)) * 8
                take = max(8, min(take, n))
                if n - take < 8:
                    take = n
            buckets[w].append((c, b, j0, take))
            j0 += take
            n -= take
            acc += c * take
            if acc >= per - 1e-9 and w < _NW - 1:
                w += 1
                acc = 0.0
    return buckets


_BUCKETS = _balanced_buckets()


def _chunks(spans, kind, size):
    out = []
    for c, b, j0, n in spans:
        if c != kind:
            continue
        off = 0
        while off < n:
            m = min(size, n - off)
            out.append((b, j0 + off, m))
            off += m
    return out


def _pad_body(x0, x1, x2, x3, x4, x5, x6, x7, out, zbuf, buf0, buf1,
              sem_z, sem_in, sem_out):
    xs = (x0, x1, x2, x3, x4, x5, x6, x7)
    wid = lax.axis_index("s") * _NC + lax.axis_index("c")

    def _zero_row(r, carry):
        for cc in range(_D // 16):
            zbuf[r, pl.ds(cc * 16, 16)] = jnp.zeros((16,), jnp.float32)
        return carry

    lax.fori_loop(0, _ZROWS, _zero_row, 0)

    bufs = (buf0, buf1)
    for w, spans in enumerate(_BUCKETS):
        zchunks = _chunks(spans, 1, _ZROWS)
        cchunks = _chunks(spans, 2, _CROWS)

        @pl.when(wid == w)
        def _(zchunks=zchunks, cchunks=cchunks):
            zpend = [
                pltpu.async_copy(
                    zbuf.at[pl.ds(0, m), :],
                    out.at[b, pl.ds(j0, m), :],
                    sem_z,
                )
                for b, j0, m in zchunks
            ]
            opend = [None] * len(cchunks)
            for k, (b, j0, m) in enumerate(cchunks):
                s = k % 2
                if k >= 2:
                    opend[k - 2].wait()
                ind = pltpu.async_copy(
                    xs[b].at[pl.ds(j0, m), :],
                    bufs[s].at[pl.ds(0, m), :],
                    sem_in,
                )
                ind.wait()
                opend[k] = pltpu.async_copy(
                    bufs[s].at[pl.ds(0, m), :],
                    out.at[b, pl.ds(j0, m), :],
                    sem_out,
                )
            for k in range(max(len(cchunks) - 2, 0), len(cchunks)):
                opend[k].wait()
            for p in zpend:
                p.wait()


@functools.cache
def _get_padder():
    return pl.kernel(
        _pad_body,
        out_type=jax.ShapeDtypeStruct((_B, _MAX, _D), jnp.float32),
        mesh=plsc.VectorSubcoreMesh(
            core_axis_name="c",
            subcore_axis_name="s",
            num_cores=_NC,
            num_subcores=_NS,
        ),
        scratch_types=[
            pltpu.VMEM((_ZROWS, _D), jnp.float32),
            pltpu.VMEM((_CROWS, _D), jnp.float32),
            pltpu.VMEM((_CROWS, _D), jnp.float32),
            pltpu.SemaphoreType.DMA,
            pltpu.SemaphoreType.DMA,
            pltpu.SemaphoreType.DMA,
        ],
    )


def kernel(x0, x1, x2, x3, x4, x5, x6, x7):
    return _get_padder()(x0, x1, x2, x3, x4, x5, x6, x7)


# speedup vs baseline: 16.6576x; 1.0827x over previous
import functools

import jax
import jax.numpy as jnp
from jax import lax
from jax.experimental import pallas as pl
from jax.experimental.pallas import tpu as pltpu
from jax.experimental.pallas import tpu_sc as plsc

_SEQ_LENS = (2048, 1792, 1536, 1280, 1024, 768, 512, 256)
_B = 8
_MAX = 2048
_D = 1024
_NC = 2
_NS = 16
_NW = _NC * _NS
_ZROWS = 24
_CROWS = 48


def _chunk_list():
    chunks = []
    for b, length in enumerate(_SEQ_LENS):
        for j in range(0, length, _CROWS):
            chunks.append(("c", b, j, min(_CROWS, length - j)))
        for j in range(length, _MAX, _ZROWS):
            chunks.append(("z", b, j, min(_ZROWS, _MAX - j)))
    return chunks


def _balanced_buckets():
    chunks = _chunk_list()
    total = sum(n for _, _, _, n in chunks)
    per = total / _NW
    buckets = [[] for _ in range(_NW)]
    w, acc = 0, 0
    for ch in chunks:
        while w < _NW - 1 and acc + ch[3] / 2.0 >= (w + 1) * per:
            w += 1
        buckets[w].append(ch)
        acc += ch[3]
    return buckets


_BUCKETS = _balanced_buckets()


def _pad_body(x0, x1, x2, x3, x4, x5, x6, x7, out, zbuf, buf0, buf1,
              sem_z, sem_in, sem_out):
    xs = (x0, x1, x2, x3, x4, x5, x6, x7)
    wid = lax.axis_index("s") * _NC + lax.axis_index("c")

    bufs = (buf0, buf1)
    for w, spans in enumerate(_BUCKETS):
        zchunks = [ch for ch in spans if ch[0] == "z"]
        cchunks = [ch for ch in spans if ch[0] == "c"]

        @pl.when(wid == w)
        def _(zchunks=zchunks, cchunks=cchunks):
            if zchunks:
                def _zero_row(r, carry):
                    for cc in range(_D // 16):
                        zbuf[r, pl.ds(cc * 16, 16)] = jnp.zeros(
                            (16,), jnp.float32)
                    return carry

                lax.fori_loop(0, _ZROWS, _zero_row, 0)

            zpend = [
                pltpu.async_copy(
                    zbuf.at[pl.ds(0, m), :],
                    out.at[b, pl.ds(j0, m), :],
                    sem_z,
                )
                for _, b, j0, m in zchunks
            ]
            opend = [None] * len(cchunks)
            for k, (_, b, j0, m) in enumerate(cchunks):
                s = k % 2
                if k >= 2:
                    opend[k - 2].wait()
                ind = pltpu.async_copy(
                    xs[b].at[pl.ds(j0, m), :],
                    bufs[s].at[pl.ds(0, m), :],
                    sem_in,
                )
                ind.wait()
                opend[k] = pltpu.async_copy(
                    bufs[s].at[pl.ds(0, m), :],
                    out.at[b, pl.ds(j0, m), :],
                    sem_out,
                )
            for k in range(max(len(cchunks) - 2, 0), len(cchunks)):
                opend[k].wait()
            for p in zpend:
                p.wait()


@functools.cache
def _get_padder():
    return pl.kernel(
        _pad_body,
        out_type=jax.ShapeDtypeStruct((_B, _MAX, _D), jnp.float32),
        mesh=plsc.VectorSubcoreMesh(
            core_axis_name="c",
            subcore_axis_name="s",
            num_cores=_NC,
            num_subcores=_NS,
        ),
        scratch_types=[
            pltpu.VMEM((_ZROWS, _D), jnp.float32),
            pltpu.VMEM((_CROWS, _D), jnp.float32),
            pltpu.VMEM((_CROWS, _D), jnp.float32),
            pltpu.SemaphoreType.DMA,
            pltpu.SemaphoreType.DMA,
            pltpu.SemaphoreType.DMA,
        ],
    )


def kernel(x0, x1, x2, x3, x4, x5, x6, x7):
    return _get_padder()(x0, x1, x2, x3, x4, x5, x6, x7)


# speedup vs baseline: 16.7836x; 1.0076x over previous
import functools

import jax
import jax.numpy as jnp
from jax import lax
from jax.experimental import pallas as pl
from jax.experimental.pallas import tpu as pltpu
from jax.experimental.pallas import tpu_sc as plsc

_SEQ_LENS = (2048, 1792, 1536, 1280, 1024, 768, 512, 256)
_B = 8
_MAX = 2048
_D = 1024
_NC = 2
_NS = 16
_NW = _NC * _NS
_ROWS = 120
_CROWS = 40


def _balanced_spans():
    spans = []
    for b, length in enumerate(_SEQ_LENS):
        spans.append(("c", b, 0, length))
        if length < _MAX:
            spans.append(("z", b, length, _MAX - length))
    total = sum(n for _, _, _, n in spans)
    per = total / _NW
    buckets = [[] for _ in range(_NW)]
    w, done = 0, 0
    for kind, b, j0, n in spans:
        while n > 0:
            owed = (w + 1) * per - done
            take = min(n, max(8, int(round(owed / 8.0)) * 8))
            if n - take < 8:
                take = n
            buckets[w].append((kind, b, j0, take))
            j0 += take
            n -= take
            done += take
            if done >= (w + 1) * per - 4 and w < _NW - 1:
                w += 1
    return buckets


_SPANS = _balanced_spans()


def _chunked(spans, kind, cap):
    out = []
    for k, b, j0, n in spans:
        if k != kind:
            continue
        off = 0
        while off < n:
            m = min(cap, n - off)
            out.append((b, j0 + off, m))
            off += m
    return out


def _pad_body(x0, x1, x2, x3, x4, x5, x6, x7, out, big, sem_z, sem_in,
              sem_out):
    xs = (x0, x1, x2, x3, x4, x5, x6, x7)
    wid = lax.axis_index("s") * _NC + lax.axis_index("c")

    def _fill_zero(row0, nrows):
        def _row(r, carry):
            for cc in range(_D // 16):
                big[r, pl.ds(cc * 16, 16)] = jnp.zeros((16,), jnp.float32)
            return carry

        lax.fori_loop(row0, row0 + nrows, _row, 0)

    for w, spans in enumerate(_SPANS):
        has_copy = any(s[0] == "c" for s in spans)
        has_zero = any(s[0] == "z" for s in spans)

        @pl.when(wid == w)
        def _(spans=spans, has_copy=has_copy, has_zero=has_zero):
            zpend = []
            if has_zero and not has_copy:
                zchunks = _chunked(spans, "z", _ROWS)
                first = zchunks[0]
                m0 = min(40, first[2])
                _fill_zero(0, m0)
                zpend.append(
                    pltpu.async_copy(
                        big.at[pl.ds(0, m0), :],
                        out.at[first[0], pl.ds(first[1], m0), :],
                        sem_z,
                    )
                )
                _fill_zero(m0, _ROWS - m0)
                rest = [(first[0], first[1] + m0, first[2] - m0)] \
                    if first[2] > m0 else []
                rest += zchunks[1:]
                for b, j0, m in rest:
                    zpend.append(
                        pltpu.async_copy(
                            big.at[pl.ds(0, m), :],
                            out.at[b, pl.ds(j0, m), :],
                            sem_z,
                        )
                    )
            elif has_zero:
                _fill_zero(2 * _CROWS, _ROWS - 2 * _CROWS)
                for b, j0, m in _chunked(spans, "z", _ROWS - 2 * _CROWS):
                    zpend.append(
                        pltpu.async_copy(
                            big.at[pl.ds(2 * _CROWS, m), :],
                            out.at[b, pl.ds(j0, m), :],
                            sem_z,
                        )
                    )

            if has_copy:
                nbuf = 2 if has_zero else 3
                cchunks = _chunked(spans, "c", _CROWS)
                opend = [None] * len(cchunks)
                for k, (b, j0, m) in enumerate(cchunks):
                    base = (k % nbuf) * _CROWS
                    if k >= nbuf:
                        opend[k - nbuf].wait()
                    ind = pltpu.async_copy(
                        xs[b].at[pl.ds(j0, m), :],
                        big.at[pl.ds(base, m), :],
                        sem_in,
                    )
                    ind.wait()
                    opend[k] = pltpu.async_copy(
                        big.at[pl.ds(base, m), :],
                        out.at[b, pl.ds(j0, m), :],
                        sem_out,
                    )
                for k in range(max(len(cchunks) - nbuf, 0), len(cchunks)):
                    opend[k].wait()
            for p in zpend:
                p.wait()


@functools.cache
def _get_padder():
    return pl.kernel(
        _pad_body,
        out_type=jax.ShapeDtypeStruct((_B, _MAX, _D), jnp.float32),
        mesh=plsc.VectorSubcoreMesh(
            core_axis_name="c",
            subcore_axis_name="s",
            num_cores=_NC,
            num_subcores=_NS,
        ),
        scratch_types=[
            pltpu.VMEM((_ROWS, _D), jnp.float32),
            pltpu.SemaphoreType.DMA,
            pltpu.SemaphoreType.DMA,
            pltpu.SemaphoreType.DMA,
        ],
    )


def kernel(x0, x1, x2, x3, x4, x5, x6, x7):
    return _get_padder()(x0, x1, x2, x3, x4, x5, x6, x7)


# speedup vs baseline: 19.0962x; 1.1378x over previous
import functools

import jax
import jax.numpy as jnp
from jax import lax
from jax.experimental import pallas as pl
from jax.experimental.pallas import tpu as pltpu
from jax.experimental.pallas import tpu_sc as plsc

_SEQ_LENS = (2048, 1792, 1536, 1280, 1024, 768, 512, 256)
_B = 8
_MAX = 2048
_D = 1024
_NC = 2
_NS = 16
_NW = _NC * _NS
_WROWS = _B * _MAX // _NW
_ROWS = 120
_CR = 40


def _pad_body(x0, x1, x2, x3, x4, x5, x6, x7, out, big, sem_z, sem_in,
              sem_out):
    xs = (x0, x1, x2, x3, x4, x5, x6, x7)
    wid = lax.axis_index("s") * _NC + lax.axis_index("c")
    bb = wid // 4
    q = wid % 4
    jlo = q * _WROWS

    def _fill_zero(row0, nrows):
        def _row(r, carry):
            for cc in range(_D // 16):
                big[r, pl.ds(cc * 16, 16)] = jnp.zeros((16,), jnp.float32)
            return carry

        lax.fori_loop(row0, row0 + nrows, _row, 0)

    def _drain(sem, nrows):
        pltpu.make_async_copy(
            x0.at[pl.ds(0, nrows), :], big.at[pl.ds(0, nrows), :], sem
        ).wait()

    nfull = 12
    for b, length in enumerate(_SEQ_LENS):
        nworkers = length // _WROWS
        if nworkers == 0:
            continue

        @pl.when((bb == b) & (q < nworkers))
        def _(b=b):
            def _step(k, carry):
                base = lax.rem(k, 3) * _CR
                j = jlo + k * _CR

                @pl.when(k >= 3)
                def _():
                    _drain(sem_out, _CR)

                pltpu.async_copy(
                    xs[b].at[pl.ds(j, _CR), :],
                    big.at[pl.ds(base, _CR), :],
                    sem_in,
                ).wait()
                pltpu.async_copy(
                    big.at[pl.ds(base, _CR), :],
                    out.at[b, pl.ds(j, _CR), :],
                    sem_out,
                )
                return carry

            lax.fori_loop(0, nfull, _step, 0)
            _drain(sem_out, _CR)
            j = jlo + nfull * _CR
            pltpu.async_copy(
                xs[b].at[pl.ds(j, 32), :], big.at[pl.ds(0, 32), :], sem_in
            ).wait()
            pltpu.async_copy(
                big.at[pl.ds(0, 32), :], out.at[b, pl.ds(j, 32), :], sem_out
            )
            _drain(sem_out, _WROWS - 10 * _CR)

    @pl.when(q * _WROWS + 256 * bb >= _MAX)
    def _():
        _fill_zero(0, _CR)
        pltpu.async_copy(
            big.at[pl.ds(0, _CR), :],
            out.at[bb, pl.ds(jlo, _CR), :],
            sem_z,
        )
        _fill_zero(_CR, _ROWS - _CR)
        for t in range(3):
            pltpu.async_copy(
                big.at[pl.ds(0, _ROWS), :],
                out.at[bb, pl.ds(jlo + _CR + t * _ROWS, _ROWS), :],
                sem_z,
            )
        rem = _WROWS - _CR - 3 * _ROWS
        pltpu.async_copy(
            big.at[pl.ds(0, rem), :],
            out.at[bb, pl.ds(jlo + _WROWS - rem, rem), :],
            sem_z,
        )
        _drain(sem_z, _CR)
        for t in range(3):
            _drain(sem_z, _ROWS)
        _drain(sem_z, rem)

    for b, length in enumerate(_SEQ_LENS):
        if length % _WROWS == 0:
            continue
        w = 4 * b + length // _WROWS
        j0 = (length // _WROWS) * _WROWS

        @pl.when(wid == w)
        def _(b=b, j0=j0):
            chunks = [(j0 + t * _CR, _CR) for t in range(6)] + [
                (j0 + 6 * _CR, 16)
            ]
            opend = [None] * len(chunks)
            for k, (j, m) in enumerate(chunks):
                base = (k % 2) * _CR
                if k >= 2:
                    opend[k - 2].wait()
                pltpu.async_copy(
                    xs[b].at[pl.ds(j, m), :],
                    big.at[pl.ds(base, m), :],
                    sem_in,
                ).wait()
                opend[k] = pltpu.async_copy(
                    big.at[pl.ds(base, m), :],
                    out.at[b, pl.ds(j, m), :],
                    sem_out,
                )
            _fill_zero(2 * _CR, _ROWS - 2 * _CR)
            zpend = []
            z0 = j0 + 256
            zchunks = [(z0 + t * _CR, _CR) for t in range(6)] + [
                (z0 + 6 * _CR, 16)
            ]
            for j, m in zchunks:
                zpend.append(
                    pltpu.async_copy(
                        big.at[pl.ds(2 * _CR, m), :],
                        out.at[b, pl.ds(j, m), :],
                        sem_z,
                    )
                )
            for p in opend[-2:]:
                p.wait()
            for p in zpend:
                p.wait()


@functools.cache
def _get_padder():
    return pl.kernel(
        _pad_body,
        out_type=jax.ShapeDtypeStruct((_B, _MAX, _D), jnp.float32),
        mesh=plsc.VectorSubcoreMesh(
            core_axis_name="c",
            subcore_axis_name="s",
            num_cores=_NC,
            num_subcores=_NS,
        ),
        scratch_types=[
            pltpu.VMEM((_ROWS, _D), jnp.float32),
            pltpu.SemaphoreType.DMA,
            pltpu.SemaphoreType.DMA,
            pltpu.SemaphoreType.DMA,
        ],
    )


def kernel(x0, x1, x2, x3, x4, x5, x6, x7):
    return _get_padder()(x0, x1, x2, x3, x4, x5, x6, x7)
